# baseline (device time: 16729 ns/iter reference)
import jax
import jax.numpy as jnp
from jax import lax
from jax.experimental import pallas as pl
from jax.experimental.pallas import tpu as pltpu

N_DEV = 16


def kernel(x, W, labels):
    t, d = x.shape
    _, v_loc = W.shape
    labels2d = labels.reshape(t, 1)

    def body(x_ref, w_ref, lab_ref, out_ref, comm_ref, send_sems, recv_sems):
        my = lax.axis_index("i")

        logits = jnp.dot(
            x_ref[:, :], w_ref[:, :], preferred_element_type=jnp.float32
        )
        m_loc = jnp.max(logits, axis=1, keepdims=True)
        s_loc = jnp.sum(jnp.exp(logits - m_loc), axis=1, keepdims=True)
        col = lax.broadcasted_iota(jnp.int32, (t, v_loc), 1)
        mask = (col + my * v_loc) == lab_ref[:, :]
        c_loc = jnp.sum(jnp.where(mask, logits, 0.0), axis=1, keepdims=True)
        payload = jnp.concatenate([m_loc, s_loc, c_loc], axis=1).T
        comm_ref[pl.ds(my, 1)] = payload[None, :, :]

        bar = pltpu.get_barrier_semaphore()
        for p in range(N_DEV):
            @pl.when(my != p)
            def _(p=p):
                pl.semaphore_signal(
                    bar, inc=1,
                    device_id=(p,), device_id_type=pl.DeviceIdType.MESH,
                )
        pl.semaphore_wait(bar, N_DEV - 1)

        def send_desc(p):
            return pltpu.make_async_remote_copy(
                src_ref=comm_ref.at[my],
                dst_ref=comm_ref.at[my],
                send_sem=send_sems.at[p],
                recv_sem=recv_sems.at[my],
                device_id=(p,),
                device_id_type=pl.DeviceIdType.MESH,
            )

        for p in range(N_DEV):
            @pl.when(my != p)
            def _(p=p):
                send_desc(p).start()

        for p in range(N_DEV):
            @pl.when(my != p)
            def _(p=p):
                recv = pltpu.make_async_remote_copy(
                    src_ref=comm_ref.at[p],
                    dst_ref=comm_ref.at[p],
                    send_sem=send_sems.at[p],
                    recv_sem=recv_sems.at[p],
                    device_id=(p,),
                    device_id_type=pl.DeviceIdType.MESH,
                )
                recv.wait_recv()

        allp = comm_ref[:, :, :]
        m_all = allp[:, 0, :]
        s_all = allp[:, 1, :]
        c_all = allp[:, 2, :]
        m_g = jnp.max(m_all, axis=0, keepdims=True)
        s_g = jnp.sum(s_all * jnp.exp(m_all - m_g), axis=0, keepdims=True)
        c_g = jnp.sum(c_all, axis=0, keepdims=True)
        out_ref[:, :] = m_g + jnp.log(s_g) - c_g

        for p in range(N_DEV):
            @pl.when(my != p)
            def _(p=p):
                send_desc(p).wait_send()

    out = pl.pallas_call(
        body,
        out_shape=jax.ShapeDtypeStruct((1, t), jnp.float32),
        in_specs=[
            pl.BlockSpec(memory_space=pltpu.VMEM),
            pl.BlockSpec(memory_space=pltpu.VMEM),
            pl.BlockSpec(memory_space=pltpu.VMEM),
        ],
        out_specs=pl.BlockSpec(memory_space=pltpu.VMEM),
        scratch_shapes=[
            pltpu.VMEM((N_DEV, 3, t), jnp.float32),
            pltpu.SemaphoreType.DMA((N_DEV,)),
            pltpu.SemaphoreType.DMA((N_DEV,)),
        ],
        compiler_params=pltpu.CompilerParams(collective_id=0),
    )(x, W, labels2d)
    return out.reshape(t)


# device time: 15969 ns/iter; 1.0476x vs baseline; 1.0476x over previous
import jax
import jax.numpy as jnp
from jax import lax
from jax.experimental import pallas as pl
from jax.experimental.pallas import tpu as pltpu

N_DEV = 16


def kernel(x, W, labels):
    t, d = x.shape
    _, v_loc = W.shape
    labels2d = labels.reshape(t, 1)

    def body(x_ref, w_ref, lab_ref, out_ref, comm_ref, send_sems, recv_sems):
        my = lax.axis_index("i")

        bar = pltpu.get_barrier_semaphore()
        for p in range(N_DEV):
            @pl.when(my != p)
            def _(p=p):
                pl.semaphore_signal(
                    bar, inc=1,
                    device_id=(p,), device_id_type=pl.DeviceIdType.MESH,
                )

        logits = jnp.dot(
            x_ref[:, :].astype(jnp.bfloat16),
            w_ref[:, :].astype(jnp.bfloat16),
            preferred_element_type=jnp.float32,
        )
        m_loc = jnp.max(logits, axis=1, keepdims=True)
        s_loc = jnp.sum(jnp.exp(logits - m_loc), axis=1, keepdims=True)
        col = lax.broadcasted_iota(jnp.int32, (t, v_loc), 1)
        mask = (col + my * v_loc) == lab_ref[:, :]
        c_loc = jnp.sum(jnp.where(mask, logits, 0.0), axis=1, keepdims=True)
        payload = jnp.concatenate([m_loc, s_loc, c_loc], axis=1).T
        comm_ref[pl.ds(my, 1)] = payload[None, :, :]

        pl.semaphore_wait(bar, N_DEV - 1)

        def send_desc(p):
            return pltpu.make_async_remote_copy(
                src_ref=comm_ref.at[my],
                dst_ref=comm_ref.at[my],
                send_sem=send_sems.at[p],
                recv_sem=recv_sems.at[my],
                device_id=(p,),
                device_id_type=pl.DeviceIdType.MESH,
            )

        for p in range(N_DEV):
            @pl.when(my != p)
            def _(p=p):
                send_desc(p).start()

        for p in range(N_DEV):
            @pl.when(my != p)
            def _(p=p):
                recv = pltpu.make_async_remote_copy(
                    src_ref=comm_ref.at[p],
                    dst_ref=comm_ref.at[p],
                    send_sem=send_sems.at[p],
                    recv_sem=recv_sems.at[p],
                    device_id=(p,),
                    device_id_type=pl.DeviceIdType.MESH,
                )
                recv.wait_recv()

        allp = comm_ref[:, :, :]
        m_all = allp[:, 0, :]
        s_all = allp[:, 1, :]
        c_all = allp[:, 2, :]
        m_g = jnp.max(m_all, axis=0, keepdims=True)
        s_g = jnp.sum(s_all * jnp.exp(m_all - m_g), axis=0, keepdims=True)
        c_g = jnp.sum(c_all, axis=0, keepdims=True)
        out_ref[:, :] = m_g + jnp.log(s_g) - c_g

        for p in range(N_DEV):
            @pl.when(my != p)
            def _(p=p):
                send_desc(p).wait_send()

    out = pl.pallas_call(
        body,
        out_shape=jax.ShapeDtypeStruct((1, t), jnp.float32),
        in_specs=[
            pl.BlockSpec(memory_space=pltpu.VMEM),
            pl.BlockSpec(memory_space=pltpu.VMEM),
            pl.BlockSpec(memory_space=pltpu.VMEM),
        ],
        out_specs=pl.BlockSpec(memory_space=pltpu.VMEM),
        scratch_shapes=[
            pltpu.VMEM((N_DEV, 3, t), jnp.float32),
            pltpu.SemaphoreType.DMA((N_DEV,)),
            pltpu.SemaphoreType.DMA((N_DEV,)),
        ],
        compiler_params=pltpu.CompilerParams(collective_id=0),
    )(x, W, labels2d)
    return out.reshape(t)
